# Initial kernel scaffold; baseline (speedup 1.0000x reference)
import jax
import jax.numpy as jnp
from jax.experimental import pallas as pl


def kernel(x, edge_index, edge_attr, batch, emlp_w0, emlp_b0, emlp_w1, emlp_b1, nn_root, nn_bias, gat_w, gat_att_src, gat_att_dst, gat_bias, fc1_w, fc1_b, fc2_w, fc2_b):
    raise NotImplementedError("write your pallas kernel here")


# speedup vs baseline: 1.1762x; 1.1762x over previous
import functools

import jax
import jax.numpy as jnp
from jax.experimental import pallas as pl

N = 10000
E = 160000
D_IN = 128
D_EDGE = 16
NN_OUT = 8
EDGE_HID = 64
GAT_OUT = 64
HID = 128
G = 64

BE = 2000
BN = 2000


def _msg_body(ea_ref, xs_ref, w0_ref, b0_ref, wq_ref, b1r_ref, out_ref):
    ea = ea_ref[...]
    xs = xs_ref[...]
    h = jnp.maximum(
        jax.lax.dot_general(ea, w0_ref[...], (((1,), (0,)), ((), ())),
                            preferred_element_type=jnp.float32) + b0_ref[...],
        0.0)
    a2 = jax.lax.dot_general(xs, wq_ref[...], (((1,), (0,)), ((), ())),
                             preferred_element_type=jnp.float32)
    parts = []
    for o in range(NN_OUT):
        blk = a2[:, o * EDGE_HID:(o + 1) * EDGE_HID]
        parts.append(jnp.sum(h * blk, axis=1, keepdims=True))
    bias = jax.lax.dot_general(xs, b1r_ref[...], (((1,), (0,)), ((), ())),
                               preferred_element_type=jnp.float32)
    out_ref[...] = jnp.concatenate(parts, axis=1) + bias


def _node_body(x_ref, agg_ref, nn_root_ref, nnb_ref, gw_ref, asrc_ref, adst_ref,
               hg_ref, a2_ref):
    x1 = jnp.maximum(
        jax.lax.dot_general(x_ref[...], nn_root_ref[...], (((1,), (0,)), ((), ())),
                            preferred_element_type=jnp.float32)
        + agg_ref[...] + nnb_ref[...], 0.0)
    hg = jax.lax.dot_general(x1, gw_ref[...], (((1,), (0,)), ((), ())),
                             preferred_element_type=jnp.float32)
    hg_ref[...] = hg
    a_src = jnp.sum(hg * asrc_ref[...], axis=1, keepdims=True)
    a_dst = jnp.sum(hg * adst_ref[...], axis=1, keepdims=True)
    a2_ref[...] = jnp.concatenate([a_src, a_dst], axis=1)


def _pool_head_body(x2_ref, batch_ref, gb_ref, f1w_ref, f1b_ref, f2w_ref, f2b_ref,
                    out_ref):
    x2 = jnp.maximum(x2_ref[...] + gb_ref[...], 0.0)
    gid = jax.lax.broadcasted_iota(jnp.int32, (G, 1), 0)
    oht = (batch_ref[...] == gid).astype(jnp.float32)
    sums = jax.lax.dot_general(oht, x2, (((1,), (0,)), ((), ())),
                               preferred_element_type=jnp.float32)
    cnt = jnp.sum(oht, axis=1, keepdims=True)
    pooled = sums / jnp.maximum(cnt, 1.0)
    z = jnp.maximum(
        jax.lax.dot_general(pooled, f1w_ref[...], (((1,), (0,)), ((), ())),
                            preferred_element_type=jnp.float32) + f1b_ref[...],
        0.0)
    out_ref[...] = jax.lax.dot_general(z, f2w_ref[...], (((1,), (0,)), ((), ())),
                                       preferred_element_type=jnp.float32) + f2b_ref[...]


def kernel(x, edge_index, edge_attr, batch, emlp_w0, emlp_b0, emlp_w1, emlp_b1,
           nn_root, nn_bias, gat_w, gat_att_src, gat_att_dst, gat_bias,
           fc1_w, fc1_b, fc2_w, fc2_b):
    src = edge_index[0]
    dst = edge_index[1]

    wq2 = jnp.transpose(emlp_w1.reshape(EDGE_HID, D_IN, NN_OUT), (1, 2, 0)) \
             .reshape(D_IN, NN_OUT * EDGE_HID)
    b1r = emlp_b1.reshape(D_IN, NN_OUT)

    xs = x[src]

    msg = pl.pallas_call(
        _msg_body,
        grid=(E // BE,),
        in_specs=[
            pl.BlockSpec((BE, D_EDGE), lambda i: (i, 0)),
            pl.BlockSpec((BE, D_IN), lambda i: (i, 0)),
            pl.BlockSpec((D_EDGE, EDGE_HID), lambda i: (0, 0)),
            pl.BlockSpec((1, EDGE_HID), lambda i: (0, 0)),
            pl.BlockSpec((D_IN, NN_OUT * EDGE_HID), lambda i: (0, 0)),
            pl.BlockSpec((D_IN, NN_OUT), lambda i: (0, 0)),
        ],
        out_specs=pl.BlockSpec((BE, NN_OUT), lambda i: (i, 0)),
        out_shape=jax.ShapeDtypeStruct((E, NN_OUT), jnp.float32),
    )(edge_attr, xs, emlp_w0, emlp_b0.reshape(1, EDGE_HID), wq2, b1r)

    agg = jax.ops.segment_sum(msg, dst, num_segments=N)

    hg, a2 = pl.pallas_call(
        _node_body,
        grid=(N // BN,),
        in_specs=[
            pl.BlockSpec((BN, D_IN), lambda i: (i, 0)),
            pl.BlockSpec((BN, NN_OUT), lambda i: (i, 0)),
            pl.BlockSpec((D_IN, NN_OUT), lambda i: (0, 0)),
            pl.BlockSpec((1, NN_OUT), lambda i: (0, 0)),
            pl.BlockSpec((NN_OUT, GAT_OUT), lambda i: (0, 0)),
            pl.BlockSpec((1, GAT_OUT), lambda i: (0, 0)),
            pl.BlockSpec((1, GAT_OUT), lambda i: (0, 0)),
        ],
        out_specs=[
            pl.BlockSpec((BN, GAT_OUT), lambda i: (i, 0)),
            pl.BlockSpec((BN, 2), lambda i: (i, 0)),
        ],
        out_shape=[
            jax.ShapeDtypeStruct((N, GAT_OUT), jnp.float32),
            jax.ShapeDtypeStruct((N, 2), jnp.float32),
        ],
    )(x, agg, nn_root, nn_bias.reshape(1, NN_OUT), gat_w,
      gat_att_src.reshape(1, GAT_OUT), gat_att_dst.reshape(1, GAT_OUT))

    a_src = a2[:, 0]
    a_dst = a2[:, 1]

    loop = jnp.arange(N, dtype=src.dtype)
    s2 = jnp.concatenate([src, loop])
    d2 = jnp.concatenate([dst, loop])
    e = jax.nn.leaky_relu(a_src[s2] + a_dst[d2], negative_slope=0.2)
    m = jax.ops.segment_max(e, d2, num_segments=N)
    ex = jnp.exp(e - m[d2])
    denom = jax.ops.segment_sum(ex, d2, num_segments=N)
    alpha = ex / (denom[d2] + 1e-16)
    x2raw = jax.ops.segment_sum(alpha[:, None] * hg[s2], d2, num_segments=N)

    out = pl.pallas_call(
        _pool_head_body,
        in_specs=[
            pl.BlockSpec((N, GAT_OUT), lambda: (0, 0)),
            pl.BlockSpec((1, N), lambda: (0, 0)),
            pl.BlockSpec((1, GAT_OUT), lambda: (0, 0)),
            pl.BlockSpec((GAT_OUT, HID), lambda: (0, 0)),
            pl.BlockSpec((1, HID), lambda: (0, 0)),
            pl.BlockSpec((HID, 1), lambda: (0, 0)),
            pl.BlockSpec((1, 1), lambda: (0, 0)),
        ],
        out_specs=pl.BlockSpec((G, 1), lambda: (0, 0)),
        out_shape=jax.ShapeDtypeStruct((G, 1), jnp.float32),
    )(x2raw, batch.reshape(1, N), gat_bias.reshape(1, GAT_OUT),
      fc1_w, fc1_b.reshape(1, HID), fc2_w, fc2_b.reshape(1, 1))

    return out[:, 0]


# speedup vs baseline: 1.5568x; 1.3235x over previous
import functools

import jax
import jax.numpy as jnp
from jax import lax
from jax.experimental import pallas as pl
from jax.experimental.pallas import tpu as pltpu
from jax.experimental.pallas import tpu_sc as plsc

N = 10000
E = 160000
D_IN = 128
D_EDGE = 16
NN_OUT = 8
EDGE_HID = 64
GAT_OUT = 64
HID = 128
G = 64

BE = 2000
BN = 2000


def _make_sc_gather(V, D, B, chunk):
    NW = 32
    b_per_w = B // NW
    assert B % (8 * NW) == 0 and b_per_w % chunk == 0 and chunk % 8 == 0
    mesh = plsc.VectorSubcoreMesh(core_axis_name="c", subcore_axis_name="s")

    @functools.partial(
        pl.kernel, mesh=mesh,
        out_type=jax.ShapeDtypeStruct((B, D), jnp.float32),
        scratch_types=[
            pltpu.VMEM((chunk,), jnp.int32),
            pltpu.VMEM((chunk, D), jnp.float32),
            pltpu.SemaphoreType.DMA,
        ],
    )
    def k(table_hbm, idx_hbm, out_hbm, idx_v, rows_v, sem):
        wid = lax.axis_index("s") * 2 + lax.axis_index("c")
        for c in range(b_per_w // chunk):
            base = wid * b_per_w + c * chunk
            pltpu.sync_copy(idx_hbm.at[pl.ds(base, chunk)], idx_v)
            pltpu.async_copy(table_hbm.at[idx_v], rows_v, sem).wait()
            pltpu.sync_copy(rows_v, out_hbm.at[pl.ds(base, chunk)])

    return k


def _msg_body(ea_ref, xs_ref, w0_ref, b0_ref, wq_ref, b1r_ref, out_ref):
    ea = ea_ref[...]
    xs = xs_ref[...]
    h = jnp.maximum(
        jax.lax.dot_general(ea, w0_ref[...], (((1,), (0,)), ((), ())),
                            preferred_element_type=jnp.float32) + b0_ref[...],
        0.0)
    a2 = jax.lax.dot_general(xs, wq_ref[...], (((1,), (0,)), ((), ())),
                             preferred_element_type=jnp.float32)
    parts = []
    for o in range(NN_OUT):
        blk = a2[:, o * EDGE_HID:(o + 1) * EDGE_HID]
        parts.append(jnp.sum(h * blk, axis=1, keepdims=True))
    bias = jax.lax.dot_general(xs, b1r_ref[...], (((1,), (0,)), ((), ())),
                               preferred_element_type=jnp.float32)
    out_ref[...] = jnp.concatenate(parts, axis=1) + bias


def _node_body(x_ref, agg_ref, nn_root_ref, nnb_ref, gw_ref, asrc_ref, adst_ref,
               hgp_ref):
    x1 = jnp.maximum(
        jax.lax.dot_general(x_ref[...], nn_root_ref[...], (((1,), (0,)), ((), ())),
                            preferred_element_type=jnp.float32)
        + agg_ref[...] + nnb_ref[...], 0.0)
    hg = jax.lax.dot_general(x1, gw_ref[...], (((1,), (0,)), ((), ())),
                             preferred_element_type=jnp.float32)
    a_src = jnp.sum(hg * asrc_ref[...], axis=1, keepdims=True)
    a_dst = jnp.sum(hg * adst_ref[...], axis=1, keepdims=True)
    pad = jnp.zeros((hg.shape[0], D_IN - GAT_OUT - 2), jnp.float32)
    hgp_ref[...] = jnp.concatenate([hg, a_src, a_dst, pad], axis=1)


def _pool_head_body(x2_ref, batch_ref, gb_ref, f1w_ref, f1b_ref, f2w_ref, f2b_ref,
                    out_ref):
    x2 = jnp.maximum(x2_ref[...] + gb_ref[...], 0.0)
    gid = jax.lax.broadcasted_iota(jnp.int32, (G, 1), 0)
    oht = (batch_ref[...] == gid).astype(jnp.float32)
    sums = jax.lax.dot_general(oht, x2, (((1,), (0,)), ((), ())),
                               preferred_element_type=jnp.float32)
    cnt = jnp.sum(oht, axis=1, keepdims=True)
    pooled = sums / jnp.maximum(cnt, 1.0)
    z = jnp.maximum(
        jax.lax.dot_general(pooled, f1w_ref[...], (((1,), (0,)), ((), ())),
                            preferred_element_type=jnp.float32) + f1b_ref[...],
        0.0)
    out_ref[...] = jax.lax.dot_general(z, f2w_ref[...], (((1,), (0,)), ((), ())),
                                       preferred_element_type=jnp.float32) + f2b_ref[...]


def kernel(x, edge_index, edge_attr, batch, emlp_w0, emlp_b0, emlp_w1, emlp_b1,
           nn_root, nn_bias, gat_w, gat_att_src, gat_att_dst, gat_bias,
           fc1_w, fc1_b, fc2_w, fc2_b):
    src = edge_index[0]
    dst = edge_index[1]

    wq2 = jnp.transpose(emlp_w1.reshape(EDGE_HID, D_IN, NN_OUT), (1, 2, 0)) \
             .reshape(D_IN, NN_OUT * EDGE_HID)
    b1r = emlp_b1.reshape(D_IN, NN_OUT)

    xs = _make_sc_gather(N, D_IN, E, 1000)(x, src)

    msg = pl.pallas_call(
        _msg_body,
        grid=(E // BE,),
        in_specs=[
            pl.BlockSpec((BE, D_EDGE), lambda i: (i, 0)),
            pl.BlockSpec((BE, D_IN), lambda i: (i, 0)),
            pl.BlockSpec((D_EDGE, EDGE_HID), lambda i: (0, 0)),
            pl.BlockSpec((1, EDGE_HID), lambda i: (0, 0)),
            pl.BlockSpec((D_IN, NN_OUT * EDGE_HID), lambda i: (0, 0)),
            pl.BlockSpec((D_IN, NN_OUT), lambda i: (0, 0)),
        ],
        out_specs=pl.BlockSpec((BE, NN_OUT), lambda i: (i, 0)),
        out_shape=jax.ShapeDtypeStruct((E, NN_OUT), jnp.float32),
    )(edge_attr, xs, emlp_w0, emlp_b0.reshape(1, EDGE_HID), wq2, b1r)

    agg = jax.ops.segment_sum(msg, dst, num_segments=N)

    hgp = pl.pallas_call(
        _node_body,
        grid=(N // BN,),
        in_specs=[
            pl.BlockSpec((BN, D_IN), lambda i: (i, 0)),
            pl.BlockSpec((BN, NN_OUT), lambda i: (i, 0)),
            pl.BlockSpec((D_IN, NN_OUT), lambda i: (0, 0)),
            pl.BlockSpec((1, NN_OUT), lambda i: (0, 0)),
            pl.BlockSpec((NN_OUT, GAT_OUT), lambda i: (0, 0)),
            pl.BlockSpec((1, GAT_OUT), lambda i: (0, 0)),
            pl.BlockSpec((1, GAT_OUT), lambda i: (0, 0)),
        ],
        out_specs=pl.BlockSpec((BN, D_IN), lambda i: (i, 0)),
        out_shape=jax.ShapeDtypeStruct((N, D_IN), jnp.float32),
    )(x, agg, nn_root, nn_bias.reshape(1, NN_OUT), gat_w,
      gat_att_src.reshape(1, GAT_OUT), gat_att_dst.reshape(1, GAT_OUT))

    hg = hgp[:, :GAT_OUT]
    a_src = hgp[:, GAT_OUT]
    a_dst = hgp[:, GAT_OUT + 1]

    gsrc = _make_sc_gather(N, D_IN, E, 1000)(hgp, src)
    hg_src = gsrc[:, :GAT_OUT]
    a_src_e = gsrc[:, GAT_OUT]

    e_edge = jax.nn.leaky_relu(a_src_e + a_dst[dst], negative_slope=0.2)
    e_self = jax.nn.leaky_relu(a_src + a_dst, negative_slope=0.2)
    m = jnp.maximum(jax.ops.segment_max(e_edge, dst, num_segments=N), e_self)
    ex_edge = jnp.exp(e_edge - m[dst])
    ex_self = jnp.exp(e_self - m)
    denom = jax.ops.segment_sum(ex_edge, dst, num_segments=N) + ex_self
    alpha_edge = ex_edge / (denom[dst] + 1e-16)
    x2raw = (jax.ops.segment_sum(alpha_edge[:, None] * hg_src, dst, num_segments=N)
             + (ex_self / (denom + 1e-16))[:, None] * hg)

    out = pl.pallas_call(
        _pool_head_body,
        in_specs=[
            pl.BlockSpec((N, GAT_OUT), lambda: (0, 0)),
            pl.BlockSpec((1, N), lambda: (0, 0)),
            pl.BlockSpec((1, GAT_OUT), lambda: (0, 0)),
            pl.BlockSpec((GAT_OUT, HID), lambda: (0, 0)),
            pl.BlockSpec((1, HID), lambda: (0, 0)),
            pl.BlockSpec((HID, 1), lambda: (0, 0)),
            pl.BlockSpec((1, 1), lambda: (0, 0)),
        ],
        out_specs=pl.BlockSpec((G, 1), lambda: (0, 0)),
        out_shape=jax.ShapeDtypeStruct((G, 1), jnp.float32),
    )(x2raw, batch.reshape(1, N), gat_bias.reshape(1, GAT_OUT),
      fc1_w, fc1_b.reshape(1, HID), fc2_w, fc2_b.reshape(1, 1))

    return out[:, 0]


# speedup vs baseline: 3.2410x; 2.0819x over previous
import functools

import jax
import jax.numpy as jnp
from jax import lax
from jax.experimental import pallas as pl
from jax.experimental.pallas import tpu as pltpu
from jax.experimental.pallas import tpu_sc as plsc

N = 10000
E = 160000
D_IN = 128
D_EDGE = 16
NN_OUT = 8
EDGE_HID = 64
GAT_OUT = 64
HID = 128
G = 64

BE = 2000
BN = 2000


def _make_sc_gather(V, D, B, chunk):
    NW = 32
    b_per_w = B // NW
    assert B % (8 * NW) == 0 and b_per_w % chunk == 0 and chunk % 8 == 0
    mesh = plsc.VectorSubcoreMesh(core_axis_name="c", subcore_axis_name="s")

    @functools.partial(
        pl.kernel, mesh=mesh,
        out_type=jax.ShapeDtypeStruct((B, D), jnp.float32),
        scratch_types=[
            pltpu.VMEM((chunk,), jnp.int32),
            pltpu.VMEM((chunk, D), jnp.float32),
            pltpu.SemaphoreType.DMA,
        ],
    )
    def k(table_hbm, idx_hbm, out_hbm, idx_v, rows_v, sem):
        wid = lax.axis_index("s") * 2 + lax.axis_index("c")
        for c in range(b_per_w // chunk):
            base = wid * b_per_w + c * chunk
            pltpu.sync_copy(idx_hbm.at[pl.ds(base, chunk)], idx_v)
            pltpu.async_copy(table_hbm.at[idx_v], rows_v, sem).wait()
            pltpu.sync_copy(rows_v, out_hbm.at[pl.ds(base, chunk)])

    return k


def _msg_body(ea_ref, xs_ref, w0_ref, b0_ref, wq_ref, b1r_ref, out_ref):
    ea = ea_ref[...]
    xs = xs_ref[...]
    h = jnp.maximum(
        jax.lax.dot_general(ea, w0_ref[...], (((1,), (0,)), ((), ())),
                            preferred_element_type=jnp.float32) + b0_ref[...],
        0.0)
    a2 = jax.lax.dot_general(xs, wq_ref[...], (((1,), (0,)), ((), ())),
                             preferred_element_type=jnp.float32)
    parts = []
    for o in range(NN_OUT):
        blk = a2[:, o * EDGE_HID:(o + 1) * EDGE_HID]
        parts.append(jnp.sum(h * blk, axis=1, keepdims=True))
    bias = jax.lax.dot_general(xs, b1r_ref[...], (((1,), (0,)), ((), ())),
                               preferred_element_type=jnp.float32)
    out_ref[...] = jnp.concatenate(parts, axis=1) + bias


def _node_body(x_ref, agg_ref, nn_root_ref, nnb_ref, gw_ref, asrc_ref, adst_ref,
               hgp_ref):
    x1 = jnp.maximum(
        jax.lax.dot_general(x_ref[...], nn_root_ref[...], (((1,), (0,)), ((), ())),
                            preferred_element_type=jnp.float32)
        + agg_ref[...] + nnb_ref[...], 0.0)
    hg = jax.lax.dot_general(x1, gw_ref[...], (((1,), (0,)), ((), ())),
                             preferred_element_type=jnp.float32)
    a_src = jnp.sum(hg * asrc_ref[...], axis=1, keepdims=True)
    a_dst = jnp.sum(hg * adst_ref[...], axis=1, keepdims=True)
    pad = jnp.zeros((hg.shape[0], D_IN - GAT_OUT - 2), jnp.float32)
    hgp_ref[...] = jnp.concatenate([hg, a_src, a_dst, pad], axis=1)


def _pool_head_body(scat_ref, hg_ref, exs_ref, den_ref, batch_ref, gb_ref,
                    f1w_ref, f1b_ref, f2w_ref, f2b_ref, out_ref):
    x2u = scat_ref[...] + exs_ref[...] * hg_ref[...]
    x2n = x2u / (den_ref[...] + 1e-16)
    x2 = jnp.maximum(x2n + gb_ref[...], 0.0)
    gid = jax.lax.broadcasted_iota(jnp.int32, (G, 1), 0)
    oht = (batch_ref[...] == gid).astype(jnp.float32)
    sums = jax.lax.dot_general(oht, x2, (((1,), (0,)), ((), ())),
                               preferred_element_type=jnp.float32)
    cnt = jnp.sum(oht, axis=1, keepdims=True)
    pooled = sums / jnp.maximum(cnt, 1.0)
    z = jnp.maximum(
        jax.lax.dot_general(pooled, f1w_ref[...], (((1,), (0,)), ((), ())),
                            preferred_element_type=jnp.float32) + f1b_ref[...],
        0.0)
    out_ref[...] = jax.lax.dot_general(z, f2w_ref[...], (((1,), (0,)), ((), ())),
                                       preferred_element_type=jnp.float32) + f2b_ref[...]


def kernel(x, edge_index, edge_attr, batch, emlp_w0, emlp_b0, emlp_w1, emlp_b1,
           nn_root, nn_bias, gat_w, gat_att_src, gat_att_dst, gat_bias,
           fc1_w, fc1_b, fc2_w, fc2_b):
    src = edge_index[0]
    dst = edge_index[1]

    wq2 = jnp.transpose(emlp_w1.reshape(EDGE_HID, D_IN, NN_OUT), (1, 2, 0)) \
             .reshape(D_IN, NN_OUT * EDGE_HID)
    b1r = emlp_b1.reshape(D_IN, NN_OUT)

    xs = _make_sc_gather(N, D_IN, E, 1000)(x, src)

    msg = pl.pallas_call(
        _msg_body,
        grid=(E // BE,),
        in_specs=[
            pl.BlockSpec((BE, D_EDGE), lambda i: (i, 0)),
            pl.BlockSpec((BE, D_IN), lambda i: (i, 0)),
            pl.BlockSpec((D_EDGE, EDGE_HID), lambda i: (0, 0)),
            pl.BlockSpec((1, EDGE_HID), lambda i: (0, 0)),
            pl.BlockSpec((D_IN, NN_OUT * EDGE_HID), lambda i: (0, 0)),
            pl.BlockSpec((D_IN, NN_OUT), lambda i: (0, 0)),
        ],
        out_specs=pl.BlockSpec((BE, NN_OUT), lambda i: (i, 0)),
        out_shape=jax.ShapeDtypeStruct((E, NN_OUT), jnp.float32),
    )(edge_attr, xs, emlp_w0, emlp_b0.reshape(1, EDGE_HID), wq2, b1r)

    agg = jax.ops.segment_sum(msg, dst, num_segments=N)

    hgp = pl.pallas_call(
        _node_body,
        grid=(N // BN,),
        in_specs=[
            pl.BlockSpec((BN, D_IN), lambda i: (i, 0)),
            pl.BlockSpec((BN, NN_OUT), lambda i: (i, 0)),
            pl.BlockSpec((D_IN, NN_OUT), lambda i: (0, 0)),
            pl.BlockSpec((1, NN_OUT), lambda i: (0, 0)),
            pl.BlockSpec((NN_OUT, GAT_OUT), lambda i: (0, 0)),
            pl.BlockSpec((1, GAT_OUT), lambda i: (0, 0)),
            pl.BlockSpec((1, GAT_OUT), lambda i: (0, 0)),
        ],
        out_specs=pl.BlockSpec((BN, D_IN), lambda i: (i, 0)),
        out_shape=jax.ShapeDtypeStruct((N, D_IN), jnp.float32),
    )(x, agg, nn_root, nn_bias.reshape(1, NN_OUT), gat_w,
      gat_att_src.reshape(1, GAT_OUT), gat_att_dst.reshape(1, GAT_OUT))

    hg = hgp[:, :GAT_OUT]
    a_src = hgp[:, GAT_OUT]
    a_dst = hgp[:, GAT_OUT + 1]

    gsrc = _make_sc_gather(N, D_IN, E, 1000)(hgp, src)
    hg_src = gsrc[:, :GAT_OUT]
    a_src_e = gsrc[:, GAT_OUT]

    a_dst_e = _make_sc_gather(N, D_IN, E, 1000)(hgp, dst)[:, GAT_OUT + 1]
    e_edge = jax.nn.leaky_relu(a_src_e + a_dst_e, negative_slope=0.2)
    e_self = jax.nn.leaky_relu(a_src + a_dst, negative_slope=0.2)
    m = jnp.maximum(jax.ops.segment_max(e_edge, dst, num_segments=N), e_self)
    mt = jnp.pad(m.reshape(N, 1), ((0, 0), (0, D_IN - 1)))
    m_e = _make_sc_gather(N, D_IN, E, 1000)(mt, dst)[:, 0]
    ex_edge = jnp.exp(e_edge - m_e)
    ex_self = jnp.exp(e_self - m)
    denom = jax.ops.segment_sum(ex_edge, dst, num_segments=N) + ex_self
    scat = jax.ops.segment_sum(ex_edge[:, None] * hg_src, dst, num_segments=N)

    out = pl.pallas_call(
        _pool_head_body,
        in_specs=[
            pl.BlockSpec((N, GAT_OUT), lambda: (0, 0)),
            pl.BlockSpec((N, GAT_OUT), lambda: (0, 0)),
            pl.BlockSpec((N, 1), lambda: (0, 0)),
            pl.BlockSpec((N, 1), lambda: (0, 0)),
            pl.BlockSpec((1, N), lambda: (0, 0)),
            pl.BlockSpec((1, GAT_OUT), lambda: (0, 0)),
            pl.BlockSpec((GAT_OUT, HID), lambda: (0, 0)),
            pl.BlockSpec((1, HID), lambda: (0, 0)),
            pl.BlockSpec((HID, 1), lambda: (0, 0)),
            pl.BlockSpec((1, 1), lambda: (0, 0)),
        ],
        out_specs=pl.BlockSpec((G, 1), lambda: (0, 0)),
        out_shape=jax.ShapeDtypeStruct((G, 1), jnp.float32),
    )(scat, hg, ex_self.reshape(N, 1), denom.reshape(N, 1),
      batch.reshape(1, N), gat_bias.reshape(1, GAT_OUT),
      fc1_w, fc1_b.reshape(1, HID), fc2_w, fc2_b.reshape(1, 1))

    return out[:, 0]
